# baseline (speedup 1.0000x reference)
import functools

import jax
import jax.numpy as jnp
from jax import lax
from jax.experimental import pallas as pl
from jax.experimental.pallas import tpu as pltpu
from jax.experimental.pallas import tpu_sc as plsc

VOCAB = 100000
D = 64
B = 4096
L = 200
N = B * L

NC = 2
NS = 16
NW = NC * NS
BW = B // NW
DB = D // 8


def _transpose_tile(l, gbuf, obuf, pos_v):
    def j_body(j, carry):
        lanes = jnp.full((16,), l, jnp.int32)
        for dd in range(8):
            d = j * 8 + dd
            cols = jnp.full((16,), d, jnp.int32)
            p = plsc.load_gather(pos_v, [lanes, cols])
            for cv in range(8):
                rows = lax.iota(jnp.int32, 16) + cv * 16
                g = plsc.load_gather(gbuf, [rows, cols])
                obuf[j, dd, pl.ds(cv * 16, 16)] = g + g + p
        return carry

    lax.fori_loop(0, DB, j_body, 0, unroll=False)


def _sc_kernel(tok_hbm, table_hbm, pos_hbm, out_hbm,
               idx_v, g0, g1, o0, o1, pos_v, sem_g, sem_s):
    wid = lax.axis_index("s") * NC + lax.axis_index("c")

    pltpu.sync_copy(pos_hbm, pos_v)
    pltpu.sync_copy(tok_hbm.at[:, pl.ds(wid * BW, BW)], idx_v)

    def gather(l, gbuf):
        pltpu.async_copy(table_hbm.at[idx_v.at[l]], gbuf, sem_g)

    def wait_gather():
        pltpu.make_async_copy(table_hbm.at[idx_v.at[0]], g0, sem_g).wait()

    def scatter(l, obuf):
        pltpu.async_copy(obuf, out_hbm.at[l, :, wid], sem_s)

    def wait_scatter():
        pltpu.make_async_copy(o0, out_hbm.at[0, :, wid], sem_s).wait()

    gather(0, g0)

    def body(i, carry):
        l0 = 2 * i
        l1 = l0 + 1
        gather(l1, g1)
        wait_gather()

        @pl.when(i > 0)
        def _():
            wait_scatter()

        _transpose_tile(l0, g0, o0, pos_v)
        scatter(l0, o0)

        @pl.when(i < L // 2 - 1)
        def _():
            gather(l0 + 2, g0)

        wait_gather()

        @pl.when(i > 0)
        def _():
            wait_scatter()

        _transpose_tile(l1, g1, o1, pos_v)
        scatter(l1, o1)
        return carry

    lax.fori_loop(0, L // 2, body, 0, unroll=False)

    wait_scatter()
    wait_scatter()


def kernel(tokens, table, pos_emb):
    tok_t = tokens.T.astype(jnp.int32)
    mesh = plsc.VectorSubcoreMesh(core_axis_name="c", subcore_axis_name="s")
    run = functools.partial(
        pl.kernel,
        mesh=mesh,
        out_type=jax.ShapeDtypeStruct((L, DB, NW, 8, 128), jnp.float32),
        scratch_types=[
            pltpu.VMEM((L, BW), jnp.int32),
            pltpu.VMEM((BW, D), jnp.float32),
            pltpu.VMEM((BW, D), jnp.float32),
            pltpu.VMEM((DB, 8, 128), jnp.float32),
            pltpu.VMEM((DB, 8, 128), jnp.float32),
            pltpu.VMEM((L, D), jnp.float32),
            pltpu.SemaphoreType.DMA,
            pltpu.SemaphoreType.DMA,
        ],
        compiler_params=pltpu.CompilerParams(use_tc_tiling_on_sc=False,
                                             needs_layout_passes=False),
    )(_sc_kernel)
    out5 = run(tok_t, table, pos_emb)
    return out5.transpose(2, 4, 0, 1, 3).reshape(B, L, D)


# speedup vs baseline: 1.4387x; 1.4387x over previous
import functools

import jax
import jax.numpy as jnp
from jax import lax
from jax.experimental import pallas as pl
from jax.experimental.pallas import tpu as pltpu
from jax.experimental.pallas import tpu_sc as plsc

VOCAB = 100000
D = 64
B = 4096
L = 200
N = B * L

NC = 2
NS = 16
NW = NC * NS
BW = B // NW
DB = D // 8


def _transpose_tile(l, gbuf, obuf, pos_v):
    lanes = jnp.full((16,), l, jnp.int32)

    @plsc.parallel_loop(0, DB, 1, unroll=2)
    def j_body(j):
        for dd in range(8):
            d = j * 8 + dd
            cols = jnp.full((16,), d, jnp.int32)
            p = plsc.load_gather(pos_v, [lanes, cols])
            vals = []
            for cv in range(8):
                rows = lax.iota(jnp.int32, 16) + cv * 16
                vals.append(plsc.load_gather(gbuf, [rows, cols]))
            for cv in range(8):
                g = vals[cv]
                obuf[j, dd, pl.ds(cv * 16, 16)] = g + g + p


def _sc_kernel(tok_hbm, table_hbm, pos_hbm, out_hbm,
               idx_v, g0, g1, o0, o1, pos_v, sem_g, sem_s):
    wid = lax.axis_index("s") * NC + lax.axis_index("c")

    pltpu.sync_copy(pos_hbm, pos_v)
    pltpu.sync_copy(tok_hbm.at[:, pl.ds(wid * BW, BW)], idx_v)

    def gather(l, gbuf):
        pltpu.async_copy(table_hbm.at[idx_v.at[l]], gbuf, sem_g)

    def wait_gather():
        pltpu.make_async_copy(table_hbm.at[idx_v.at[0]], g0, sem_g).wait()

    def scatter(l, obuf):
        pltpu.async_copy(obuf, out_hbm.at[l, :, wid], sem_s)

    def wait_scatter():
        pltpu.make_async_copy(o0, out_hbm.at[0, :, wid], sem_s).wait()

    gather(0, g0)

    def body(i, carry):
        l0 = 2 * i
        l1 = l0 + 1
        gather(l1, g1)
        wait_gather()

        @pl.when(i > 0)
        def _():
            wait_scatter()

        _transpose_tile(l0, g0, o0, pos_v)
        scatter(l0, o0)

        @pl.when(i < L // 2 - 1)
        def _():
            gather(l0 + 2, g0)

        wait_gather()

        @pl.when(i > 0)
        def _():
            wait_scatter()

        _transpose_tile(l1, g1, o1, pos_v)
        scatter(l1, o1)
        return carry

    lax.fori_loop(0, L // 2, body, 0, unroll=False)

    wait_scatter()
    wait_scatter()


def kernel(tokens, table, pos_emb):
    tok_t = tokens.T.astype(jnp.int32)
    mesh = plsc.VectorSubcoreMesh(core_axis_name="c", subcore_axis_name="s")
    run = functools.partial(
        pl.kernel,
        mesh=mesh,
        out_type=jax.ShapeDtypeStruct((L, DB, NW, 8, 128), jnp.float32),
        scratch_types=[
            pltpu.VMEM((L, BW), jnp.int32),
            pltpu.VMEM((BW, D), jnp.float32),
            pltpu.VMEM((BW, D), jnp.float32),
            pltpu.VMEM((DB, 8, 128), jnp.float32),
            pltpu.VMEM((DB, 8, 128), jnp.float32),
            pltpu.VMEM((L, D), jnp.float32),
            pltpu.SemaphoreType.DMA,
            pltpu.SemaphoreType.DMA,
        ],
        compiler_params=pltpu.CompilerParams(use_tc_tiling_on_sc=False,
                                             needs_layout_passes=False),
    )(_sc_kernel)
    out5 = run(tok_t, table, pos_emb)
    return out5.transpose(2, 4, 0, 1, 3).reshape(B, L, D)


# speedup vs baseline: 1.7342x; 1.2054x over previous
import functools

import jax
import jax.numpy as jnp
from jax import lax
from jax.experimental import pallas as pl
from jax.experimental.pallas import tpu as pltpu
from jax.experimental.pallas import tpu_sc as plsc

VOCAB = 100000
D = 64
B = 4096
L = 200
N = B * L

NC = 2
NS = 16
NW = NC * NS
BW = B // NW
DB = D // 8


def _transpose_tile(l, gbuf, obuf, pos_v):
    lanes = jnp.full((16,), l, jnp.int32)
    iota = lax.iota(jnp.int32, 16)

    @plsc.parallel_loop(0, D // 16, 1)
    def db_body(db):
        for k in range(16):
            dvec = db * 16 + ((iota + k) & 15)
            p = plsc.load_gather(pos_v, [lanes, dvec])
            jv = dvec >> 3
            ddv = dvec & 7
            for cb in range(8):
                rows = iota + cb * 16
                g = plsc.load_gather(gbuf, [rows, dvec])
                plsc.store_scatter(obuf, [jv, ddv, rows], g + g + p)


def _sc_kernel(tok_hbm, table_hbm, pos_hbm, out_hbm,
               idx_v, g0, g1, o0, o1, pos_v, sem_g, sem_s):
    wid = lax.axis_index("s") * NC + lax.axis_index("c")

    pltpu.sync_copy(pos_hbm, pos_v)
    pltpu.sync_copy(tok_hbm.at[:, pl.ds(wid * BW, BW)], idx_v)

    def gather(l, gbuf):
        pltpu.async_copy(table_hbm.at[idx_v.at[l]], gbuf, sem_g)

    def wait_gather():
        pltpu.make_async_copy(table_hbm.at[idx_v.at[0]], g0, sem_g).wait()

    def scatter(l, obuf):
        pltpu.async_copy(obuf, out_hbm.at[l, :, wid], sem_s)

    def wait_scatter():
        pltpu.make_async_copy(o0, out_hbm.at[0, :, wid], sem_s).wait()

    gather(0, g0)

    def body(i, carry):
        l0 = 2 * i
        l1 = l0 + 1
        gather(l1, g1)
        wait_gather()

        @pl.when(i > 0)
        def _():
            wait_scatter()

        _transpose_tile(l0, g0, o0, pos_v)
        scatter(l0, o0)

        @pl.when(i < L // 2 - 1)
        def _():
            gather(l0 + 2, g0)

        wait_gather()

        @pl.when(i > 0)
        def _():
            wait_scatter()

        _transpose_tile(l1, g1, o1, pos_v)
        scatter(l1, o1)
        return carry

    lax.fori_loop(0, L // 2, body, 0, unroll=False)

    wait_scatter()
    wait_scatter()


def kernel(tokens, table, pos_emb):
    tok_t = tokens.T.astype(jnp.int32)
    mesh = plsc.VectorSubcoreMesh(core_axis_name="c", subcore_axis_name="s")
    run = functools.partial(
        pl.kernel,
        mesh=mesh,
        out_type=jax.ShapeDtypeStruct((L, DB, NW, 8, 128), jnp.float32),
        scratch_types=[
            pltpu.VMEM((L, BW), jnp.int32),
            pltpu.VMEM((BW, D), jnp.float32),
            pltpu.VMEM((BW, D), jnp.float32),
            pltpu.VMEM((DB, 8, 128), jnp.float32),
            pltpu.VMEM((DB, 8, 128), jnp.float32),
            pltpu.VMEM((L, D), jnp.float32),
            pltpu.SemaphoreType.DMA,
            pltpu.SemaphoreType.DMA,
        ],
        compiler_params=pltpu.CompilerParams(use_tc_tiling_on_sc=False,
                                             needs_layout_passes=False),
    )(_sc_kernel)
    out5 = run(tok_t, table, pos_emb)
    return out5.transpose(2, 4, 0, 1, 3).reshape(B, L, D)


# speedup vs baseline: 1.9846x; 1.1444x over previous
import functools

import jax
import jax.numpy as jnp
from jax import lax
from jax.experimental import pallas as pl
from jax.experimental.pallas import tpu as pltpu
from jax.experimental.pallas import tpu_sc as plsc

VOCAB = 100000
D = 64
B = 4096
L = 200
N = B * L

NC = 2
NS = 16
NW = NC * NS
BW = B // NW
DB = D // 8
BLK = 4


def _sc_kernel(tok_hbm, table_hbm, pos_hbm, out_hbm,
               idx_v, gbuf, obuf, pos_v, sem_g, sem_s):
    wid = lax.axis_index("s") * NC + lax.axis_index("c")

    pltpu.sync_copy(pos_hbm, pos_v)
    pltpu.sync_copy(tok_hbm.at[:, pl.ds(wid * BW, BW)], idx_v)

    def fire_block(l0):
        half = (l0 // BLK) % 2
        for k in range(BLK):
            pltpu.async_copy(table_hbm.at[idx_v.at[l0 + k]],
                             gbuf.at[pl.ds((half * BLK + k) * BW, BW)],
                             sem_g)

    def wait_block():
        for _ in range(BLK):
            pltpu.make_async_copy(table_hbm.at[idx_v.at[0]],
                                  gbuf.at[pl.ds(0, BW)], sem_g).wait()

    def wait_scatter():
        pltpu.make_async_copy(obuf.at[0], out_hbm.at[0, :, wid], sem_s).wait()

    fire_block(0)

    iota = lax.iota(jnp.int32, 16)

    def body(l, carry):
        @pl.when(l % BLK == 0)
        def _():
            wait_block()

            @pl.when(l + BLK < L)
            def _():
                fire_block(l + BLK)

        @pl.when(l >= 2)
        def _():
            wait_scatter()

        par = l % 2
        row0 = (((l // BLK) % 2) * BLK + (l % BLK)) * BW
        lanes = jnp.full((16,), l, jnp.int32)
        pvec = jnp.full((16,), par, jnp.int32)

        @plsc.parallel_loop(0, D // 16, 1)
        def db_body(db):
            for k in range(16):
                dvec = db * 16 + ((iota + k) & 15)
                p = plsc.load_gather(pos_v, [lanes, dvec])
                jv = dvec >> 3
                ddv = dvec & 7
                for cb in range(8):
                    rows = iota + cb * 16
                    g = plsc.load_gather(gbuf, [row0 + rows, dvec])
                    plsc.store_scatter(obuf, [pvec, jv, ddv, rows], g + g + p)

        pltpu.async_copy(obuf.at[par], out_hbm.at[l, :, wid], sem_s)
        return carry

    lax.fori_loop(0, L, body, 0, unroll=False)

    wait_scatter()
    wait_scatter()


def kernel(tokens, table, pos_emb):
    tok_t = tokens.T.astype(jnp.int32)
    mesh = plsc.VectorSubcoreMesh(core_axis_name="c", subcore_axis_name="s")
    run = functools.partial(
        pl.kernel,
        mesh=mesh,
        out_type=jax.ShapeDtypeStruct((L, DB, NW, 8, 128), jnp.float32),
        scratch_types=[
            pltpu.VMEM((L, BW), jnp.int32),
            pltpu.VMEM((2 * BLK * BW, D), jnp.float32),
            pltpu.VMEM((2, DB, 8, 128), jnp.float32),
            pltpu.VMEM((L, D), jnp.float32),
            pltpu.SemaphoreType.DMA,
            pltpu.SemaphoreType.DMA,
        ],
        compiler_params=pltpu.CompilerParams(use_tc_tiling_on_sc=False,
                                             needs_layout_passes=False),
    )(_sc_kernel)
    out5 = run(tok_t, table, pos_emb)
    return out5.transpose(2, 4, 0, 1, 3).reshape(B, L, D)
